# baseline (device time: 110112 ns/iter reference)
import jax
import jax.numpy as jnp
from jax import lax
from jax.experimental import pallas as pl
from jax.experimental.pallas import tpu as pltpu

N_DEV = 4
SQ = 1024
SKV = 1024
DM = 1024
HDM = DM // 2
HL = 8
DH = 128
SCALE = 0.08838834764831843

_G_OFF = (0, 3, 1, 2)

QC = 256
_PIECES = {
    0: ((0, 1024),),
    1: ((0, 640),),
    2: ((0, 128), (384, 512)),
    3: ((0, 128), (640, 384)),
}

_sem_signal = getattr(pltpu, "semaphore_signal", None) or pl.semaphore_signal
_sem_wait = getattr(pltpu, "semaphore_wait", None) or pl.semaphore_wait
_CompilerParams = getattr(pltpu, "CompilerParams", None) or getattr(
    pltpu, "TPUCompilerParams"
)


def kernel(x, Wq, K_ext, V_ext, Wo):
    def body(
        x_ref, wq_ref, k_hbm, v_hbm, wo_ref, out_ref,
        cL, cR, c2, s1, rL, rR, s2, r2, kv_sems,
        k_scr, v_scr, q_scr, ctx_scr, bias_scr,
    ):
        me = lax.axis_index("i")
        right = lax.rem(me + 1, N_DEV)
        left = lax.rem(me + N_DEV - 1, N_DEV)

        def kv_copies(t):
            g = lax.rem(me + _G_OFF[t], N_DEV)
            cps = []
            for h in range(HL):
                gh = g * HL + h
                cps.append(pltpu.make_async_copy(
                    k_hbm.at[me, :, gh, :], k_scr.at[t % 2, h],
                    kv_sems.at[t % 2]))
                cps.append(pltpu.make_async_copy(
                    v_hbm.at[me, :, gh, :], v_scr.at[t % 2, h],
                    kv_sems.at[t % 2]))
            return cps

        for cp in kv_copies(0):
            cp.start()

        qi = lax.broadcasted_iota(jnp.int32, (SQ, SKV), 0)
        ki = lax.broadcasted_iota(jnp.int32, (SQ, SKV), 1)
        mask = (jnp.abs(qi - ki) <= 128) | (ki < 32) | (qi < 32)
        bias_scr[...] = jnp.where(mask, 0.0, -1e9).astype(jnp.bfloat16)

        def rdma(src, dst, send_sem, recv_sem, dev):
            return pltpu.make_async_remote_copy(
                src_ref=src, dst_ref=dst, send_sem=send_sem,
                recv_sem=recv_sem, device_id=(dev,),
                device_id_type=pl.DeviceIdType.MESH,
            )

        snd_wq_R = rdma(wq_ref, cL.at[0], s1.at[0], rL.at[0], right)
        snd_wo_R = rdma(wo_ref, cL.at[1], s1.at[1], rL.at[1], right)
        snd_wq_L = rdma(wq_ref, cR.at[0], s1.at[2], rR.at[0], left)
        snd_wo_L = rdma(wo_ref, cR.at[1], s1.at[3], rR.at[1], left)
        fwd_R = rdma(cL.at[0], c2.at[0], s2.at[0], r2.at[0], right)
        fwd_L = rdma(cR.at[1], c2.at[1], s2.at[1], r2.at[1], left)
        hop1 = (snd_wq_R, snd_wo_R, snd_wq_L, snd_wo_L)

        barrier = pltpu.get_barrier_semaphore()
        for nbr in (left, right):
            _sem_signal(
                barrier, inc=1, device_id=(nbr,),
                device_id_type=pl.DeviceIdType.MESH,
            )
        _sem_wait(barrier, 2)

        for d in hop1:
            d.start()

        def attn_stage(t, wq_src, wo_src, pre_q=None, pre_o=None):
            slot = t % 2
            if t + 1 < N_DEV:
                for cp in kv_copies(t + 1):
                    cp.start()
            if pre_q is not None:
                pre_q()
            for cp in kv_copies(t):
                cp.wait()
            q = jnp.dot(
                x_ref[0], wq_src[...], preferred_element_type=jnp.float32
            )
            q_scr[...] = (q * SCALE).astype(jnp.bfloat16)
            for h in range(HL):
                k_h = k_scr[slot, h].astype(jnp.bfloat16)
                v_h = v_scr[slot, h].astype(jnp.bfloat16)
                for r in range(SQ // QC):
                    q_c = q_scr[r * QC:(r + 1) * QC, h * DH:(h + 1) * DH]
                    ctx_acc = None
                    d_acc = None
                    for lo, ln in _PIECES[r]:
                        s = lax.dot_general(
                            q_c, k_h[lo:lo + ln], (((1,), (1,)), ((), ())),
                            preferred_element_type=jnp.float32,
                        )
                        p = jnp.exp(s + bias_scr[r * QC:(r + 1) * QC,
                                                 lo:lo + ln])
                        dp = jnp.sum(p, axis=1, keepdims=True)
                        cp = jnp.dot(
                            p.astype(jnp.bfloat16), v_h[lo:lo + ln],
                            preferred_element_type=jnp.float32,
                        )
                        ctx_acc = cp if ctx_acc is None else ctx_acc + cp
                        d_acc = dp if d_acc is None else d_acc + dp
                    ctx_c = ctx_acc * (1.0 / d_acc)
                    ctx_scr[r * QC:(r + 1) * QC, h * DH:(h + 1) * DH] = (
                        ctx_c.astype(jnp.bfloat16))
            if pre_o is not None:
                pre_o()
            part = jnp.dot(
                ctx_scr[...], wo_src[...], preferred_element_type=jnp.float32
            )
            if t == 0:
                out_ref[0, :, :] = part
            else:
                out_ref[0, :, :] = out_ref[0, :, :] + part

        attn_stage(0, wq_ref, wo_ref)
        attn_stage(
            1, cL.at[0], cL.at[1],
            pre_q=lambda: (snd_wq_R.wait_recv(), fwd_R.start()),
            pre_o=snd_wo_R.wait_recv,
        )
        attn_stage(
            2, cR.at[0], cR.at[1],
            pre_q=snd_wq_L.wait_recv,
            pre_o=lambda: (snd_wo_L.wait_recv(), fwd_L.start()),
        )
        attn_stage(
            3, c2.at[0], c2.at[1],
            pre_q=fwd_R.wait_recv,
            pre_o=fwd_L.wait_recv,
        )

        for d in hop1 + (fwd_R, fwd_L):
            d.wait_send()

    out = pl.pallas_call(
        body,
        out_shape=jax.ShapeDtypeStruct((1, SQ, DM), jnp.float32),
        in_specs=[
            pl.BlockSpec(memory_space=pltpu.MemorySpace.VMEM),
            pl.BlockSpec(memory_space=pltpu.MemorySpace.VMEM),
            pl.BlockSpec(memory_space=pl.ANY),
            pl.BlockSpec(memory_space=pl.ANY),
            pl.BlockSpec(memory_space=pltpu.MemorySpace.VMEM),
        ],
        out_specs=pl.BlockSpec(memory_space=pltpu.MemorySpace.VMEM),
        scratch_shapes=[
            pltpu.VMEM((2, DM, DM), jnp.bfloat16),
            pltpu.VMEM((2, DM, DM), jnp.bfloat16),
            pltpu.VMEM((2, DM, DM), jnp.bfloat16),
            pltpu.SemaphoreType.DMA((4,)),
            pltpu.SemaphoreType.DMA((2,)),
            pltpu.SemaphoreType.DMA((2,)),
            pltpu.SemaphoreType.DMA((2,)),
            pltpu.SemaphoreType.DMA((2,)),
            pltpu.SemaphoreType.DMA((2,)),
            pltpu.VMEM((2, HL, SKV, DH), jnp.float32),
            pltpu.VMEM((2, HL, SKV, DH), jnp.float32),
            pltpu.VMEM((SQ, DM), jnp.bfloat16),
            pltpu.VMEM((SQ, DM), jnp.bfloat16),
            pltpu.VMEM((SQ, SKV), jnp.bfloat16),
        ],
        compiler_params=_CompilerParams(
            collective_id=0, vmem_limit_bytes=120 * 1024 * 1024
        ),
    )(
        x.astype(jnp.bfloat16),
        Wq.astype(jnp.bfloat16),
        K_ext,
        V_ext,
        Wo.astype(jnp.bfloat16),
    )
    return out


# device time: 106098 ns/iter; 1.0378x vs baseline; 1.0378x over previous
import jax
import jax.numpy as jnp
from jax import lax
from jax.experimental import pallas as pl
from jax.experimental.pallas import tpu as pltpu

N_DEV = 4
SQ = 1024
SKV = 1024
DM = 1024
HDM = DM // 2
HL = 8
DH = 128
SCALE = 0.08838834764831843

_G_OFF = (0, 3, 1, 2)

QC = 256
_PIECES = {
    0: ((0, 1024),),
    1: ((0, 640),),
    2: ((0, 128), (384, 512)),
    3: ((0, 128), (640, 384)),
}

_sem_signal = getattr(pltpu, "semaphore_signal", None) or pl.semaphore_signal
_sem_wait = getattr(pltpu, "semaphore_wait", None) or pl.semaphore_wait
_CompilerParams = getattr(pltpu, "CompilerParams", None) or getattr(
    pltpu, "TPUCompilerParams"
)


def kernel(x, Wq, K_ext, V_ext, Wo):
    def body(
        x_ref, wq_ref, k_hbm, v_hbm, wo_ref, out_ref,
        cL, cR, c2, s1, rL, rR, s2, r2, kv_sems,
        k_scr, v_scr, q_scr, ctx_scr, bias_scr,
    ):
        me = lax.axis_index("i")
        right = lax.rem(me + 1, N_DEV)
        left = lax.rem(me + N_DEV - 1, N_DEV)

        def kv_copies(t):
            g = lax.rem(me + _G_OFF[t], N_DEV)
            cps = []
            for h in range(HL):
                gh = g * HL + h
                cps.append(pltpu.make_async_copy(
                    k_hbm.at[me, :, gh, :], k_scr.at[t % 2, h],
                    kv_sems.at[t % 2]))
                cps.append(pltpu.make_async_copy(
                    v_hbm.at[me, :, gh, :], v_scr.at[t % 2, h],
                    kv_sems.at[t % 2]))
            return cps

        for cp in kv_copies(0):
            cp.start()

        qi = lax.broadcasted_iota(jnp.int32, (SQ, SKV), 0)
        ki = lax.broadcasted_iota(jnp.int32, (SQ, SKV), 1)
        mask = (jnp.abs(qi - ki) <= 128) | (ki < 32) | (qi < 32)
        bias_scr[...] = jnp.where(mask, 0.0, -1e9).astype(jnp.bfloat16)

        def rdma(src, dst, send_sem, recv_sem, dev):
            return pltpu.make_async_remote_copy(
                src_ref=src, dst_ref=dst, send_sem=send_sem,
                recv_sem=recv_sem, device_id=(dev,),
                device_id_type=pl.DeviceIdType.MESH,
            )

        snd_wq_R = rdma(wq_ref, cL.at[0], s1.at[0], rL.at[0], right)
        snd_wo_R = rdma(wo_ref, cL.at[1], s1.at[1], rL.at[1], right)
        snd_wq_L = rdma(wq_ref, cR.at[0], s1.at[2], rR.at[0], left)
        snd_wo_L = rdma(wo_ref, cR.at[1], s1.at[3], rR.at[1], left)
        fwd_R = rdma(cL.at[0], c2.at[0], s2.at[0], r2.at[0], right)
        fwd_L = rdma(cR.at[1], c2.at[1], s2.at[1], r2.at[1], left)
        hop1 = (snd_wq_R, snd_wo_R, snd_wq_L, snd_wo_L)

        barrier = pltpu.get_barrier_semaphore()
        for nbr in (left, right):
            _sem_signal(
                barrier, inc=1, device_id=(nbr,),
                device_id_type=pl.DeviceIdType.MESH,
            )
        _sem_wait(barrier, 2)

        for d in hop1:
            d.start()

        def attn_stage(t, wq_src, wo_src, pre_q=None, pre_o=None):
            slot = t % 2
            if t + 1 < N_DEV:
                for cp in kv_copies(t + 1):
                    cp.start()
            if pre_q is not None:
                pre_q()
            for cp in kv_copies(t):
                cp.wait()
            q = jnp.dot(
                x_ref[0], wq_src[...], preferred_element_type=jnp.float32
            )
            q_scr[...] = (q * SCALE).astype(jnp.bfloat16)
            for h in range(HL):
                k_h = k_scr[slot, h].astype(jnp.bfloat16)
                v_h = v_scr[slot, h].astype(jnp.bfloat16)
                for r in range(SQ // QC):
                    q_c = q_scr[r * QC:(r + 1) * QC, h * DH:(h + 1) * DH]
                    ctx_acc = None
                    d_acc = None
                    for lo, ln in _PIECES[r]:
                        s = lax.dot_general(
                            q_c, k_h[lo:lo + ln], (((1,), (1,)), ((), ())),
                            preferred_element_type=jnp.float32,
                        )
                        p = jnp.exp(s + bias_scr[r * QC:(r + 1) * QC,
                                                 lo:lo + ln])
                        dp = jnp.sum(p, axis=1, keepdims=True)
                        cp = jnp.dot(
                            p.astype(jnp.bfloat16), v_h[lo:lo + ln],
                            preferred_element_type=jnp.float32,
                        )
                        ctx_acc = cp if ctx_acc is None else ctx_acc + cp
                        d_acc = dp if d_acc is None else d_acc + dp
                    ctx_c = ctx_acc * (1.0 / d_acc)
                    ctx_scr[r * QC:(r + 1) * QC, h * DH:(h + 1) * DH] = (
                        ctx_c.astype(jnp.bfloat16))
            if pre_o is not None:
                pre_o()
            part = jnp.dot(
                ctx_scr[...], wo_src[...], preferred_element_type=jnp.float32
            )
            if t == 0:
                out_ref[0, :, :] = part
            else:
                out_ref[0, :, :] = out_ref[0, :, :] + part

        attn_stage(0, wq_ref, wo_ref)
        attn_stage(
            1, cL.at[0], cL.at[1],
            pre_q=lambda: (snd_wq_R.wait_recv(), fwd_R.start()),
            pre_o=lambda: (snd_wo_R.wait_recv(), snd_wo_L.wait_recv(),
                           fwd_L.start()),
        )
        attn_stage(
            2, cR.at[0], cR.at[1],
            pre_q=snd_wq_L.wait_recv,
        )
        attn_stage(
            3, c2.at[0], c2.at[1],
            pre_q=fwd_R.wait_recv,
            pre_o=fwd_L.wait_recv,
        )

        for d in hop1 + (fwd_R, fwd_L):
            d.wait_send()

    out = pl.pallas_call(
        body,
        out_shape=jax.ShapeDtypeStruct((1, SQ, DM), jnp.float32),
        in_specs=[
            pl.BlockSpec(memory_space=pltpu.MemorySpace.VMEM),
            pl.BlockSpec(memory_space=pltpu.MemorySpace.VMEM),
            pl.BlockSpec(memory_space=pl.ANY),
            pl.BlockSpec(memory_space=pl.ANY),
            pl.BlockSpec(memory_space=pltpu.MemorySpace.VMEM),
        ],
        out_specs=pl.BlockSpec(memory_space=pltpu.MemorySpace.VMEM),
        scratch_shapes=[
            pltpu.VMEM((2, DM, DM), jnp.bfloat16),
            pltpu.VMEM((2, DM, DM), jnp.bfloat16),
            pltpu.VMEM((2, DM, DM), jnp.bfloat16),
            pltpu.SemaphoreType.DMA((4,)),
            pltpu.SemaphoreType.DMA((2,)),
            pltpu.SemaphoreType.DMA((2,)),
            pltpu.SemaphoreType.DMA((2,)),
            pltpu.SemaphoreType.DMA((2,)),
            pltpu.SemaphoreType.DMA((2,)),
            pltpu.VMEM((2, HL, SKV, DH), jnp.float32),
            pltpu.VMEM((2, HL, SKV, DH), jnp.float32),
            pltpu.VMEM((SQ, DM), jnp.bfloat16),
            pltpu.VMEM((SQ, DM), jnp.bfloat16),
            pltpu.VMEM((SQ, SKV), jnp.bfloat16),
        ],
        compiler_params=_CompilerParams(
            collective_id=0, vmem_limit_bytes=120 * 1024 * 1024
        ),
    )(
        x.astype(jnp.bfloat16),
        Wq.astype(jnp.bfloat16),
        K_ext,
        V_ext,
        Wo.astype(jnp.bfloat16),
    )
    return out


# device time: 100548 ns/iter; 1.0951x vs baseline; 1.0552x over previous
import jax
import jax.numpy as jnp
from jax import lax
from jax.experimental import pallas as pl
from jax.experimental.pallas import tpu as pltpu

N_DEV = 4
SQ = 1024
SKV = 1024
DM = 1024
HW = DM // 2
HL = 8
DH = 128
SCALE = 0.08838834764831843

_G_OFF = (0, 3, 1, 2)

QC = 256
_PIECES = {
    0: ((0, 1024),),
    1: ((0, 640),),
    2: ((0, 128), (384, 512)),
    3: ((0, 128), (640, 384)),
}

_sem_signal = getattr(pltpu, "semaphore_signal", None) or pl.semaphore_signal
_sem_wait = getattr(pltpu, "semaphore_wait", None) or pl.semaphore_wait
_CompilerParams = getattr(pltpu, "CompilerParams", None) or getattr(
    pltpu, "TPUCompilerParams"
)


def kernel(x, Wq, K_ext, V_ext, Wo):
    def body(
        x_ref, wq_ref, k_hbm, v_hbm, wo_ref, out_ref,
        cL, cR, c2, s1, rL, rR, s2, r2, kv_sems,
        k_scr, v_scr, q_scr, ctx_scr, bias_scr,
    ):
        me = lax.axis_index("i")
        right = lax.rem(me + 1, N_DEV)
        left = lax.rem(me + N_DEV - 1, N_DEV)

        def kv_copies(t):
            g = lax.rem(me + _G_OFF[t], N_DEV)
            cps = []
            for h in range(HL):
                gh = g * HL + h
                cps.append(pltpu.make_async_copy(
                    k_hbm.at[me, :, gh, :], k_scr.at[t % 2, h],
                    kv_sems.at[t % 2]))
                cps.append(pltpu.make_async_copy(
                    v_hbm.at[me, :, gh, :], v_scr.at[t % 2, h],
                    kv_sems.at[t % 2]))
            return cps

        for cp in kv_copies(0):
            cp.start()

        qi = lax.broadcasted_iota(jnp.int32, (SQ, SKV), 0)
        ki = lax.broadcasted_iota(jnp.int32, (SQ, SKV), 1)
        mask = (jnp.abs(qi - ki) <= 128) | (ki < 32) | (qi < 32)
        bias_scr[...] = jnp.where(mask, 0.0, -1e9).astype(jnp.bfloat16)

        def rdma(src, dst, send_sem, recv_sem, dev):
            return pltpu.make_async_remote_copy(
                src_ref=src, dst_ref=dst, send_sem=send_sem,
                recv_sem=recv_sem, device_id=(dev,),
                device_id_type=pl.DeviceIdType.MESH,
            )

        def halves(buf_in, comm, ti, sems_s, sems_r, base, dev):
            out = []
            for j, lo in enumerate((0, HW)):
                out.append(rdma(
                    buf_in.at[:, pl.ds(lo, HW)],
                    comm.at[ti, :, pl.ds(lo, HW)],
                    sems_s.at[base + j], sems_r.at[2 * ti + j], dev))
            return out

        wqRa, wqRb = halves(wq_ref, cL, 0, s1, rL, 0, right)
        woRa, woRb = halves(wo_ref, cL, 1, s1, rL, 2, right)
        wqLa, wqLb = halves(wq_ref, cR, 0, s1, rR, 4, left)
        woLa, woLb = halves(wo_ref, cR, 1, s1, rR, 6, left)
        fwdRa = rdma(cL.at[0, :, pl.ds(0, HW)], c2.at[0, :, pl.ds(0, HW)],
                     s2.at[0], r2.at[0], right)
        fwdRb = rdma(cL.at[0, :, pl.ds(HW, HW)], c2.at[0, :, pl.ds(HW, HW)],
                     s2.at[1], r2.at[1], right)
        fwdL = rdma(cR.at[1], c2.at[1], s2.at[2], r2.at[2], left)
        hop1 = (wqRa, wqRb, woRa, woRb, wqLa, wqLb, woLa, woLb)

        barrier = pltpu.get_barrier_semaphore()
        for nbr in (left, right):
            _sem_signal(
                barrier, inc=1, device_id=(nbr,),
                device_id_type=pl.DeviceIdType.MESH,
            )
        _sem_wait(barrier, 2)

        for d in hop1:
            d.start()

        def attn_heads(slot, heads):
            for h in heads:
                k_h = k_scr[slot, h].astype(jnp.bfloat16)
                v_h = v_scr[slot, h].astype(jnp.bfloat16)
                for r in range(SQ // QC):
                    q_c = q_scr[r * QC:(r + 1) * QC, h * DH:(h + 1) * DH]
                    ctx_acc = None
                    d_acc = None
                    for lo, ln in _PIECES[r]:
                        s = lax.dot_general(
                            q_c, k_h[lo:lo + ln], (((1,), (1,)), ((), ())),
                            preferred_element_type=jnp.float32,
                        )
                        p = jnp.exp(s + bias_scr[r * QC:(r + 1) * QC,
                                                 lo:lo + ln])
                        dp = jnp.sum(p, axis=1, keepdims=True)
                        cp = jnp.dot(
                            p.astype(jnp.bfloat16), v_h[lo:lo + ln],
                            preferred_element_type=jnp.float32,
                        )
                        ctx_acc = cp if ctx_acc is None else ctx_acc + cp
                        d_acc = dp if d_acc is None else d_acc + dp
                    ctx_c = ctx_acc * (1.0 / d_acc)
                    ctx_scr[r * QC:(r + 1) * QC, h * DH:(h + 1) * DH] = (
                        ctx_c.astype(jnp.bfloat16))

        def attn_stage(t, WQ, WO, pqa=None, pqb=None, poa=None, pob=None):
            slot = t % 2
            if t + 1 < N_DEV:
                for cp in kv_copies(t + 1):
                    cp.start()
            if pqa is not None:
                pqa()
            for cp in kv_copies(t):
                cp.wait()
            q = jnp.dot(x_ref[0], WQ(0), preferred_element_type=jnp.float32)
            q_scr[:, 0:HW] = (q * SCALE).astype(jnp.bfloat16)
            attn_heads(slot, range(0, HL // 2))
            if pqb is not None:
                pqb()
            q = jnp.dot(x_ref[0], WQ(HW), preferred_element_type=jnp.float32)
            q_scr[:, HW:DM] = (q * SCALE).astype(jnp.bfloat16)
            attn_heads(slot, range(HL // 2, HL))
            if poa is not None:
                poa()
            pa = jnp.dot(
                ctx_scr[...], WO(0), preferred_element_type=jnp.float32)
            if pob is not None:
                pob()
            pb = jnp.dot(
                ctx_scr[...], WO(HW), preferred_element_type=jnp.float32)
            if t == 0:
                out_ref[0, :, 0:HW] = pa
                out_ref[0, :, HW:DM] = pb
            else:
                out_ref[0, :, 0:HW] = out_ref[0, :, 0:HW] + pa
                out_ref[0, :, HW:DM] = out_ref[0, :, HW:DM] + pb

        attn_stage(
            0,
            lambda lo: wq_ref[:, lo:lo + HW],
            lambda lo: wo_ref[:, lo:lo + HW],
        )
        attn_stage(
            1,
            lambda lo: cL[0, :, lo:lo + HW],
            lambda lo: cL[1, :, lo:lo + HW],
            pqa=lambda: (wqRa.wait_recv(), fwdRa.start()),
            pqb=lambda: (wqRb.wait_recv(), fwdRb.start()),
            poa=woRa.wait_recv,
            pob=lambda: (woRb.wait_recv(), woLa.wait_recv(),
                         woLb.wait_recv(), fwdL.start()),
        )
        attn_stage(
            2,
            lambda lo: cR[0, :, lo:lo + HW],
            lambda lo: cR[1, :, lo:lo + HW],
            pqa=wqLa.wait_recv,
            pqb=wqLb.wait_recv,
        )
        attn_stage(
            3,
            lambda lo: c2[0, :, lo:lo + HW],
            lambda lo: c2[1, :, lo:lo + HW],
            pqa=fwdRa.wait_recv,
            pqb=fwdRb.wait_recv,
            poa=fwdL.wait_recv,
        )

        for d in hop1 + (fwdRa, fwdRb, fwdL):
            d.wait_send()

    out = pl.pallas_call(
        body,
        out_shape=jax.ShapeDtypeStruct((1, SQ, DM), jnp.float32),
        in_specs=[
            pl.BlockSpec(memory_space=pltpu.MemorySpace.VMEM),
            pl.BlockSpec(memory_space=pltpu.MemorySpace.VMEM),
            pl.BlockSpec(memory_space=pl.ANY),
            pl.BlockSpec(memory_space=pl.ANY),
            pl.BlockSpec(memory_space=pltpu.MemorySpace.VMEM),
        ],
        out_specs=pl.BlockSpec(memory_space=pltpu.MemorySpace.VMEM),
        scratch_shapes=[
            pltpu.VMEM((2, DM, DM), jnp.bfloat16),
            pltpu.VMEM((2, DM, DM), jnp.bfloat16),
            pltpu.VMEM((2, DM, DM), jnp.bfloat16),
            pltpu.SemaphoreType.DMA((8,)),
            pltpu.SemaphoreType.DMA((4,)),
            pltpu.SemaphoreType.DMA((4,)),
            pltpu.SemaphoreType.DMA((3,)),
            pltpu.SemaphoreType.DMA((3,)),
            pltpu.SemaphoreType.DMA((2,)),
            pltpu.VMEM((2, HL, SKV, DH), jnp.float32),
            pltpu.VMEM((2, HL, SKV, DH), jnp.float32),
            pltpu.VMEM((SQ, DM), jnp.bfloat16),
            pltpu.VMEM((SQ, DM), jnp.bfloat16),
            pltpu.VMEM((SQ, SKV), jnp.bfloat16),
        ],
        compiler_params=_CompilerParams(
            collective_id=0, vmem_limit_bytes=120 * 1024 * 1024
        ),
    )(
        x.astype(jnp.bfloat16),
        Wq.astype(jnp.bfloat16),
        K_ext,
        V_ext,
        Wo.astype(jnp.bfloat16),
    )
    return out


# device time: 98846 ns/iter; 1.1140x vs baseline; 1.0172x over previous
import jax
import jax.numpy as jnp
from jax import lax
from jax.experimental import pallas as pl
from jax.experimental.pallas import tpu as pltpu

N_DEV = 4
SQ = 1024
SKV = 1024
DM = 1024
HW = DM // 2
HL = 8
DH = 128
SCALE = 0.08838834764831843

_G_OFF = (0, 3, 1, 2)

QC = 256
_PIECES = {
    0: ((0, 1024),),
    1: ((0, 640),),
    2: ((0, 128), (384, 512)),
    3: ((0, 128), (640, 384)),
}

_sem_signal = getattr(pltpu, "semaphore_signal", None) or pl.semaphore_signal
_sem_wait = getattr(pltpu, "semaphore_wait", None) or pl.semaphore_wait
_CompilerParams = getattr(pltpu, "CompilerParams", None) or getattr(
    pltpu, "TPUCompilerParams"
)


def kernel(x, Wq, K_ext, V_ext, Wo):
    def body(
        x_ref, wq_ref, k_hbm, v_hbm, wo_ref, out_ref,
        cL, cR, c2, s1, rL, rR, s2, r2, kv_sems,
        k_scr, v_scr, q_scr, ctx_scr,
    ):
        me = lax.axis_index("i")
        right = lax.rem(me + 1, N_DEV)
        left = lax.rem(me + N_DEV - 1, N_DEV)

        def kv_copies(t):
            g = lax.rem(me + _G_OFF[t], N_DEV)
            cps = []
            for h in range(HL):
                gh = g * HL + h
                cps.append(pltpu.make_async_copy(
                    k_hbm.at[me, :, gh, :], k_scr.at[t % 2, h],
                    kv_sems.at[t % 2]))
                cps.append(pltpu.make_async_copy(
                    v_hbm.at[me, :, gh, :], v_scr.at[t % 2, h],
                    kv_sems.at[t % 2]))
            return cps

        for cp in kv_copies(0):
            cp.start()

        def rdma(src, dst, send_sem, recv_sem, dev):
            return pltpu.make_async_remote_copy(
                src_ref=src, dst_ref=dst, send_sem=send_sem,
                recv_sem=recv_sem, device_id=(dev,),
                device_id_type=pl.DeviceIdType.MESH,
            )

        def halves(buf_in, comm, ti, sems_s, sems_r, base, dev):
            out = []
            for j, lo in enumerate((0, HW)):
                out.append(rdma(
                    buf_in.at[:, pl.ds(lo, HW)],
                    comm.at[ti, :, pl.ds(lo, HW)],
                    sems_s.at[base + j], sems_r.at[2 * ti + j], dev))
            return out

        wqRa, wqRb = halves(wq_ref, cL, 0, s1, rL, 0, right)
        woRa, woRb = halves(wo_ref, cL, 1, s1, rL, 2, right)
        wqLa, wqLb = halves(wq_ref, cR, 0, s1, rR, 4, left)
        woLa, woLb = halves(wo_ref, cR, 1, s1, rR, 6, left)
        fwdRa = rdma(cL.at[0, :, pl.ds(0, HW)], c2.at[0, :, pl.ds(0, HW)],
                     s2.at[0], r2.at[0], right)
        fwdRb = rdma(cL.at[0, :, pl.ds(HW, HW)], c2.at[0, :, pl.ds(HW, HW)],
                     s2.at[1], r2.at[1], right)
        fwdL = rdma(cR.at[1], c2.at[1], s2.at[2], r2.at[2], left)
        hop1 = (wqRa, wqRb, woRa, woRb, wqLa, wqLb, woLa, woLb)

        barrier = pltpu.get_barrier_semaphore()
        for nbr in (left, right):
            _sem_signal(
                barrier, inc=1, device_id=(nbr,),
                device_id_type=pl.DeviceIdType.MESH,
            )
        _sem_wait(barrier, 2)

        for d in hop1:
            d.start()

        def attn_heads(slot, heads):
            for h in heads:
                k_h = k_scr[slot, h].astype(jnp.bfloat16)
                v_h = v_scr[slot, h].astype(jnp.bfloat16)
                for r in range(SQ // QC):
                    q_c = q_scr[r * QC:(r + 1) * QC, h * DH:(h + 1) * DH]
                    ctx_acc = None
                    d_acc = None
                    for lo, ln in _PIECES[r]:
                        s = lax.dot_general(
                            q_c, k_h[lo:lo + ln], (((1,), (1,)), ((), ())),
                            preferred_element_type=jnp.float32,
                        )
                        qi = lax.broadcasted_iota(jnp.int32, (QC, ln), 0) + r * QC
                        ki = lax.broadcasted_iota(jnp.int32, (QC, ln), 1) + lo
                        msk = (jnp.abs(qi - ki) <= 128) | (ki < 32) | (qi < 32)
                        p = jnp.where(msk, jnp.exp(s), 0.0)
                        dp = jnp.sum(p, axis=1, keepdims=True)
                        cp = jnp.dot(
                            p.astype(jnp.bfloat16), v_h[lo:lo + ln],
                            preferred_element_type=jnp.float32,
                        )
                        ctx_acc = cp if ctx_acc is None else ctx_acc + cp
                        d_acc = dp if d_acc is None else d_acc + dp
                    ctx_c = ctx_acc * (1.0 / d_acc)
                    ctx_scr[r * QC:(r + 1) * QC, h * DH:(h + 1) * DH] = (
                        ctx_c.astype(jnp.bfloat16))

        def attn_stage(t, WQ, WO, pqa=None, pqb=None, poa=None, pob=None):
            slot = t % 2
            if t + 1 < N_DEV:
                for cp in kv_copies(t + 1):
                    cp.start()
            if pqa is not None:
                pqa()
            for cp in kv_copies(t):
                cp.wait()
            q = jnp.dot(x_ref[0], WQ(0), preferred_element_type=jnp.float32)
            q_scr[:, 0:HW] = (q * SCALE).astype(jnp.bfloat16)
            attn_heads(slot, range(0, HL // 2))
            if pqb is not None:
                pqb()
            q = jnp.dot(x_ref[0], WQ(HW), preferred_element_type=jnp.float32)
            q_scr[:, HW:DM] = (q * SCALE).astype(jnp.bfloat16)
            attn_heads(slot, range(HL // 2, HL))
            if poa is not None:
                poa()
            pa = jnp.dot(
                ctx_scr[...], WO(0), preferred_element_type=jnp.float32)
            if pob is not None:
                pob()
            pb = jnp.dot(
                ctx_scr[...], WO(HW), preferred_element_type=jnp.float32)
            if t == 0:
                out_ref[0, :, 0:HW] = pa
                out_ref[0, :, HW:DM] = pb
            else:
                out_ref[0, :, 0:HW] = out_ref[0, :, 0:HW] + pa
                out_ref[0, :, HW:DM] = out_ref[0, :, HW:DM] + pb

        attn_stage(
            0,
            lambda lo: wq_ref[:, lo:lo + HW],
            lambda lo: wo_ref[:, lo:lo + HW],
        )
        attn_stage(
            1,
            lambda lo: cL[0, :, lo:lo + HW],
            lambda lo: cL[1, :, lo:lo + HW],
            pqa=lambda: (wqRa.wait_recv(), fwdRa.start()),
            pqb=lambda: (wqRb.wait_recv(), fwdRb.start()),
            poa=woRa.wait_recv,
            pob=lambda: (woRb.wait_recv(), woLa.wait_recv(),
                         woLb.wait_recv(), fwdL.start()),
        )
        attn_stage(
            2,
            lambda lo: cR[0, :, lo:lo + HW],
            lambda lo: cR[1, :, lo:lo + HW],
            pqa=wqLa.wait_recv,
            pqb=wqLb.wait_recv,
        )
        attn_stage(
            3,
            lambda lo: c2[0, :, lo:lo + HW],
            lambda lo: c2[1, :, lo:lo + HW],
            pqa=fwdRa.wait_recv,
            pqb=fwdRb.wait_recv,
            poa=fwdL.wait_recv,
        )

        for d in hop1 + (fwdRa, fwdRb, fwdL):
            d.wait_send()

    out = pl.pallas_call(
        body,
        out_shape=jax.ShapeDtypeStruct((1, SQ, DM), jnp.float32),
        in_specs=[
            pl.BlockSpec(memory_space=pltpu.MemorySpace.VMEM),
            pl.BlockSpec(memory_space=pltpu.MemorySpace.VMEM),
            pl.BlockSpec(memory_space=pl.ANY),
            pl.BlockSpec(memory_space=pl.ANY),
            pl.BlockSpec(memory_space=pltpu.MemorySpace.VMEM),
        ],
        out_specs=pl.BlockSpec(memory_space=pltpu.MemorySpace.VMEM),
        scratch_shapes=[
            pltpu.VMEM((2, DM, DM), jnp.bfloat16),
            pltpu.VMEM((2, DM, DM), jnp.bfloat16),
            pltpu.VMEM((2, DM, DM), jnp.bfloat16),
            pltpu.SemaphoreType.DMA((8,)),
            pltpu.SemaphoreType.DMA((4,)),
            pltpu.SemaphoreType.DMA((4,)),
            pltpu.SemaphoreType.DMA((3,)),
            pltpu.SemaphoreType.DMA((3,)),
            pltpu.SemaphoreType.DMA((2,)),
            pltpu.VMEM((2, HL, SKV, DH), jnp.float32),
            pltpu.VMEM((2, HL, SKV, DH), jnp.float32),
            pltpu.VMEM((SQ, DM), jnp.bfloat16),
            pltpu.VMEM((SQ, DM), jnp.bfloat16),
        ],
        compiler_params=_CompilerParams(
            collective_id=0, vmem_limit_bytes=120 * 1024 * 1024
        ),
    )(
        x.astype(jnp.bfloat16),
        Wq.astype(jnp.bfloat16),
        K_ext,
        V_ext,
        Wo.astype(jnp.bfloat16),
    )
    return out


# device time: 92722 ns/iter; 1.1875x vs baseline; 1.0660x over previous
import jax
import jax.numpy as jnp
from jax import lax
from jax.experimental import pallas as pl
from jax.experimental.pallas import tpu as pltpu

N_DEV = 4
SQ = 1024
SKV = 1024
DM = 1024
HW = DM // 2
HL = 8
DH = 128
SCALE = 0.08838834764831843

_G_OFF = (0, 3, 1, 2)

QC = 256
_PIECES = {
    0: ((0, 1024),),
    1: ((0, 640),),
    2: ((0, 128), (384, 512)),
    3: ((0, 128), (640, 384)),
}

_sem_signal = getattr(pltpu, "semaphore_signal", None) or pl.semaphore_signal
_sem_wait = getattr(pltpu, "semaphore_wait", None) or pl.semaphore_wait
_CompilerParams = getattr(pltpu, "CompilerParams", None) or getattr(
    pltpu, "TPUCompilerParams"
)


def kernel(x, Wq, K_ext, V_ext, Wo):
    def body(
        x_ref, wq_ref, k_hbm, v_hbm, wo_ref, out_ref,
        cL, cR, c2, own, s1, rL, rR, s2, r2, kv_sems,
        k_scr, v_scr, q_scr, ctx_scr, x_bf,
    ):
        me = lax.axis_index("i")
        right = lax.rem(me + 1, N_DEV)
        left = lax.rem(me + N_DEV - 1, N_DEV)

        def kv_copies(t):
            g = lax.rem(me + _G_OFF[t], N_DEV)
            cps = []
            for h in range(HL):
                gh = g * HL + h
                cps.append(pltpu.make_async_copy(
                    k_hbm.at[me, :, gh, :], k_scr.at[t % 2, h],
                    kv_sems.at[t % 2]))
                cps.append(pltpu.make_async_copy(
                    v_hbm.at[me, :, gh, :], v_scr.at[t % 2, h],
                    kv_sems.at[t % 2]))
            return cps

        for cp in kv_copies(0):
            cp.start()

        own[0] = wq_ref[...].astype(jnp.bfloat16)
        own[1] = wo_ref[...].astype(jnp.bfloat16)
        x_bf[...] = x_ref[0].astype(jnp.bfloat16)

        def rdma(src, dst, send_sem, recv_sem, dev):
            return pltpu.make_async_remote_copy(
                src_ref=src, dst_ref=dst, send_sem=send_sem,
                recv_sem=recv_sem, device_id=(dev,),
                device_id_type=pl.DeviceIdType.MESH,
            )

        def halves(buf_in, comm, ti, sems_s, sems_r, base, dev):
            out = []
            for j, lo in enumerate((0, HW)):
                out.append(rdma(
                    buf_in.at[:, pl.ds(lo, HW)],
                    comm.at[ti, :, pl.ds(lo, HW)],
                    sems_s.at[base + j], sems_r.at[2 * ti + j], dev))
            return out

        wqRa, wqRb = halves(own.at[0], cL, 0, s1, rL, 0, right)
        woRa, woRb = halves(own.at[1], cL, 1, s1, rL, 2, right)
        wqLa, wqLb = halves(own.at[0], cR, 0, s1, rR, 4, left)
        woLa, woLb = halves(own.at[1], cR, 1, s1, rR, 6, left)
        fwdRa = rdma(cL.at[0, :, pl.ds(0, HW)], c2.at[0, :, pl.ds(0, HW)],
                     s2.at[0], r2.at[0], right)
        fwdRb = rdma(cL.at[0, :, pl.ds(HW, HW)], c2.at[0, :, pl.ds(HW, HW)],
                     s2.at[1], r2.at[1], right)
        fwdL = rdma(cR.at[1], c2.at[1], s2.at[2], r2.at[2], left)
        hop1 = (wqRa, wqRb, woRa, woRb, wqLa, wqLb, woLa, woLb)

        barrier = pltpu.get_barrier_semaphore()
        for nbr in (left, right):
            _sem_signal(
                barrier, inc=1, device_id=(nbr,),
                device_id_type=pl.DeviceIdType.MESH,
            )
        _sem_wait(barrier, 2)

        for d in hop1:
            d.start()

        def attn_heads(slot, heads):
            for h in heads:
                k_h = k_scr[slot, h].astype(jnp.bfloat16)
                v_h = v_scr[slot, h].astype(jnp.bfloat16)
                for r in range(SQ // QC):
                    q_c = q_scr[r * QC:(r + 1) * QC, h * DH:(h + 1) * DH]
                    ctx_acc = None
                    d_acc = None
                    for lo, ln in _PIECES[r]:
                        s = lax.dot_general(
                            q_c, k_h[lo:lo + ln], (((1,), (1,)), ((), ())),
                            preferred_element_type=jnp.float32,
                        )
                        qi = lax.broadcasted_iota(jnp.int32, (QC, ln), 0) + r * QC
                        ki = lax.broadcasted_iota(jnp.int32, (QC, ln), 1) + lo
                        msk = (jnp.abs(qi - ki) <= 128) | (ki < 32) | (qi < 32)
                        p = jnp.where(msk, jnp.exp(s), 0.0)
                        dp = jnp.sum(p, axis=1, keepdims=True)
                        cp = jnp.dot(
                            p.astype(jnp.bfloat16), v_h[lo:lo + ln],
                            preferred_element_type=jnp.float32,
                        )
                        ctx_acc = cp if ctx_acc is None else ctx_acc + cp
                        d_acc = dp if d_acc is None else d_acc + dp
                    ctx_c = ctx_acc * (1.0 / d_acc)
                    ctx_scr[r * QC:(r + 1) * QC, h * DH:(h + 1) * DH] = (
                        ctx_c.astype(jnp.bfloat16))

        def attn_stage(t, WQ, WO, pqa=None, pqb=None, poa=None, pob=None):
            slot = t % 2
            if t + 1 < N_DEV:
                for cp in kv_copies(t + 1):
                    cp.start()
            if pqa is not None:
                pqa()
            for cp in kv_copies(t):
                cp.wait()
            q = jnp.dot(x_bf[...], WQ(0), preferred_element_type=jnp.float32)
            q_scr[:, 0:HW] = (q * SCALE).astype(jnp.bfloat16)
            attn_heads(slot, range(0, HL // 2))
            if pqb is not None:
                pqb()
            q = jnp.dot(x_bf[...], WQ(HW), preferred_element_type=jnp.float32)
            q_scr[:, HW:DM] = (q * SCALE).astype(jnp.bfloat16)
            attn_heads(slot, range(HL // 2, HL))
            if poa is not None:
                poa()
            pa = jnp.dot(
                ctx_scr[...], WO(0), preferred_element_type=jnp.float32)
            if pob is not None:
                pob()
            pb = jnp.dot(
                ctx_scr[...], WO(HW), preferred_element_type=jnp.float32)
            if t == 0:
                out_ref[0, :, 0:HW] = pa
                out_ref[0, :, HW:DM] = pb
            else:
                out_ref[0, :, 0:HW] = out_ref[0, :, 0:HW] + pa
                out_ref[0, :, HW:DM] = out_ref[0, :, HW:DM] + pb

        attn_stage(
            0,
            lambda lo: own[0, :, lo:lo + HW],
            lambda lo: own[1, :, lo:lo + HW],
        )
        attn_stage(
            1,
            lambda lo: cL[0, :, lo:lo + HW],
            lambda lo: cL[1, :, lo:lo + HW],
            pqa=lambda: (wqRa.wait_recv(), fwdRa.start()),
            pqb=lambda: (wqRb.wait_recv(), fwdRb.start()),
            poa=woRa.wait_recv,
            pob=lambda: (woRb.wait_recv(), woLa.wait_recv(),
                         woLb.wait_recv(), fwdL.start()),
        )
        attn_stage(
            2,
            lambda lo: cR[0, :, lo:lo + HW],
            lambda lo: cR[1, :, lo:lo + HW],
            pqa=wqLa.wait_recv,
            pqb=wqLb.wait_recv,
        )
        attn_stage(
            3,
            lambda lo: c2[0, :, lo:lo + HW],
            lambda lo: c2[1, :, lo:lo + HW],
            pqa=fwdRa.wait_recv,
            pqb=fwdRb.wait_recv,
            poa=fwdL.wait_recv,
        )

        for d in hop1 + (fwdRa, fwdRb, fwdL):
            d.wait_send()

    out = pl.pallas_call(
        body,
        out_shape=jax.ShapeDtypeStruct((1, SQ, DM), jnp.float32),
        in_specs=[
            pl.BlockSpec(memory_space=pltpu.MemorySpace.VMEM),
            pl.BlockSpec(memory_space=pltpu.MemorySpace.VMEM),
            pl.BlockSpec(memory_space=pl.ANY),
            pl.BlockSpec(memory_space=pl.ANY),
            pl.BlockSpec(memory_space=pltpu.MemorySpace.VMEM),
        ],
        out_specs=pl.BlockSpec(memory_space=pltpu.MemorySpace.VMEM),
        scratch_shapes=[
            pltpu.VMEM((2, DM, DM), jnp.bfloat16),
            pltpu.VMEM((2, DM, DM), jnp.bfloat16),
            pltpu.VMEM((2, DM, DM), jnp.bfloat16),
            pltpu.VMEM((2, DM, DM), jnp.bfloat16),
            pltpu.SemaphoreType.DMA((8,)),
            pltpu.SemaphoreType.DMA((4,)),
            pltpu.SemaphoreType.DMA((4,)),
            pltpu.SemaphoreType.DMA((3,)),
            pltpu.SemaphoreType.DMA((3,)),
            pltpu.SemaphoreType.DMA((2,)),
            pltpu.VMEM((2, HL, SKV, DH), jnp.float32),
            pltpu.VMEM((2, HL, SKV, DH), jnp.float32),
            pltpu.VMEM((SQ, DM), jnp.bfloat16),
            pltpu.VMEM((SQ, DM), jnp.bfloat16),
            pltpu.VMEM((SQ, DM), jnp.bfloat16),
        ],
        compiler_params=_CompilerParams(
            collective_id=0, vmem_limit_bytes=120 * 1024 * 1024
        ),
    )(x, Wq, K_ext, V_ext, Wo)
    return out
